# baseline (speedup 1.0000x reference)
import functools

import jax
import jax.numpy as jnp
from jax.experimental import pallas as pl
from jax.experimental.pallas import tpu as pltpu

B = 8192
D = 2048
H = 64
E = 8
A = 16

CHUNK = 512
NCHUNK = B // CHUNK
NBUF = 4


def _chunk_compute(inp, act, w1, b1, w2, b2, wc, bc, wa, ba):
    acc = jnp.dot(inp.astype(jnp.bfloat16), w1,
                  preferred_element_type=jnp.float32)
    x = jnp.tanh(acc + b1)
    x = jnp.tanh(jnp.dot(x.astype(jnp.bfloat16), w2,
                         preferred_element_type=jnp.float32) + b2)
    onehot = (jax.lax.broadcasted_iota(jnp.int32, (CHUNK, E), 1) == act
              ).astype(jnp.float32)
    emask = (jax.lax.broadcasted_iota(jnp.int32, (CHUNK, E * H), 1) // H == act
             ).astype(jnp.float32)
    xb = jnp.concatenate([x] * E, axis=1) * emask
    val = (jnp.dot(xb, wc, preferred_element_type=jnp.float32)
           + jnp.dot(onehot, bc, preferred_element_type=jnp.float32))
    logits = (jnp.dot(xb, wa, preferred_element_type=jnp.float32)
              + jnp.dot(onehot, ba, preferred_element_type=jnp.float32))
    m = jnp.max(logits, axis=1, keepdims=True)
    s = logits - m
    lp = s - jnp.log(jnp.sum(jnp.exp(s), axis=1, keepdims=True))
    return val, lp


def _body(inp_hbm, act_ref, w1_ref, b1_ref, w2_ref, b2_ref,
          wc_ref, bc_ref, wa_ref, ba_ref, val_hbm, lp_hbm,
          bufs, val_v, lp_v, in_sems, out_sem):
    def in_copy(chunk, slot):
        return pltpu.make_async_copy(
            inp_hbm.at[pl.ds(chunk * CHUNK, CHUNK), :],
            bufs.at[slot], in_sems.at[slot])

    for k in range(NBUF):
        in_copy(k, k).start()

    w1 = w1_ref[...].astype(jnp.bfloat16)
    w2 = w2_ref[...].astype(jnp.bfloat16)

    for i in range(NCHUNK):
        slot = i % NBUF
        in_copy(i, slot).wait()
        act = act_ref[pl.ds(i * CHUNK, CHUNK), :]
        val, lp = _chunk_compute(bufs[slot], act, w1, b1_ref[...], w2,
                                 b2_ref[...], wc_ref[...], bc_ref[...],
                                 wa_ref[...], ba_ref[...])
        val_v[pl.ds(i * CHUNK, CHUNK), :] = val
        lp_v[pl.ds(i * CHUNK, CHUNK), :] = lp
        nxt = i + NBUF
        if nxt < NCHUNK:
            in_copy(nxt, slot).start()

    v_copy = pltpu.make_async_copy(val_v, val_hbm, out_sem)
    v_copy.start()
    l_copy = pltpu.make_async_copy(lp_v, lp_hbm, out_sem)
    l_copy.start()
    v_copy.wait()
    l_copy.wait()


@functools.partial(jax.jit, static_argnames=())
def kernel(inputs, states, masks, input_action, W1, b1, W2, b2, Wc, bc, Wa, ba):
    act2d = input_action.reshape(B, 1).astype(jnp.int32)
    wc_big = Wc.reshape(E * H, 1)
    wa_big = Wa.reshape(E * H, A)
    value, log_probs = pl.pallas_call(
        _body,
        in_specs=[
            pl.BlockSpec(memory_space=pl.ANY),
            pl.BlockSpec(memory_space=pltpu.MemorySpace.VMEM),
            pl.BlockSpec(memory_space=pltpu.MemorySpace.VMEM),
            pl.BlockSpec(memory_space=pltpu.MemorySpace.VMEM),
            pl.BlockSpec(memory_space=pltpu.MemorySpace.VMEM),
            pl.BlockSpec(memory_space=pltpu.MemorySpace.VMEM),
            pl.BlockSpec(memory_space=pltpu.MemorySpace.VMEM),
            pl.BlockSpec(memory_space=pltpu.MemorySpace.VMEM),
            pl.BlockSpec(memory_space=pltpu.MemorySpace.VMEM),
            pl.BlockSpec(memory_space=pltpu.MemorySpace.VMEM),
        ],
        out_specs=[
            pl.BlockSpec(memory_space=pl.ANY),
            pl.BlockSpec(memory_space=pl.ANY),
        ],
        out_shape=[
            jax.ShapeDtypeStruct((B, 1), jnp.float32),
            jax.ShapeDtypeStruct((B, A), jnp.float32),
        ],
        scratch_shapes=[
            pltpu.VMEM((NBUF, CHUNK, D), jnp.float32),
            pltpu.VMEM((B, 1), jnp.float32),
            pltpu.VMEM((B, A), jnp.float32),
            pltpu.SemaphoreType.DMA((NBUF,)),
            pltpu.SemaphoreType.DMA,
        ],
    )(inputs, act2d, W1, b1.reshape(1, H), W2, b2.reshape(1, H),
      wc_big, bc, wa_big, ba)
    return value, log_probs, states
